# baseline (speedup 1.0000x reference)
import jax
import jax.numpy as jnp
from jax.experimental import pallas as pl
from jax.experimental.pallas import tpu as pltpu

_MIN_PTS = 4
_NEG = -3.0e38


def _shift(x, k, fill):
    return jnp.concatenate(
        [jnp.full((k,) + x.shape[1:], fill, x.dtype), x[:-k]], axis=0)


def _fwd_kernel(kstart_ref, nkw_ref, estart_ref, new_ref,
                pts_ref, prop_ref, cu0_ref, cu1_ref, ends_ref,
                w1_ref, b1_ref, w2_ref, b2_ref,
                acc_ref, canchor_ref, ch_ref, *, blk_n, win):
    b = pl.program_id(0)

    @pl.when(b == 0)
    def _init():
        acc_ref[...] = jnp.zeros(acc_ref.shape, acc_ref.dtype)
        canchor_ref[...] = jnp.zeros(canchor_ref.shape, canchor_ref.dtype)
        ch_ref[...] = jnp.full(ch_ref.shape, _NEG, ch_ref.dtype)

    s = b * blk_n
    pts = pts_ref[...]
    pos_i = s + jax.lax.broadcasted_iota(jnp.int32, (1, blk_n), 1)

    kst0 = kstart_ref[b]

    def place_body(w, pc):
        kst = kst0 + w * win
        cu0 = cu0_ref[pl.ds(kst, win), :]
        cu1 = cu1_ref[pl.ds(kst, win), :]
        t = jnp.where(cu1 > pos_i,
                      jnp.where(cu0 == pos_i, 1.0, 0.0), 0.0)
        pw = prop_ref[pl.ds(kst, win), :]
        return pc + jax.lax.dot_general(
            t, pw, (((0,), (0,)), ((), ())),
            preferred_element_type=jnp.float32)

    pc = jax.lax.fori_loop(0, nkw_ref[b], place_body,
                           jnp.zeros((blk_n, 16), jnp.float32))

    masks = []
    a = pc[:, 0:8]
    c = pc[:, 8:9]
    k = 1
    while k < blk_n:
        mask = c == 0.0
        masks.append(mask)
        a = a + jnp.where(mask, _shift(a, k, 0.0), 0.0)
        c = c + _shift(c, k, 0.0)
        k *= 2

    first = c == 0.0
    anchors = a + jnp.where(first, 1.0, 0.0) * canchor_ref[...]

    relc = pts - anchors[:, 0:3]
    relo = pts - anchors[:, 4:7]

    w1 = w1_ref[...]
    w2 = w2_ref[...]
    b1 = b1_ref[...]
    b2 = b2_ref[...]

    def mlp(rel):
        h = jnp.maximum(jax.lax.dot_general(
            rel, w1, (((1,), (0,)), ((), ())),
            preferred_element_type=jnp.float32) + b1, 0.0)
        return jnp.maximum(jax.lax.dot_general(
            h, w2, (((1,), (0,)), ((), ())),
            preferred_element_type=jnp.float32) + b2, 0.0)

    hcat = jnp.concatenate([mlp(relc), mlp(relo)], axis=1)

    m = hcat
    k = 1
    for mask in masks:
        m = jnp.maximum(m, jnp.where(mask, _shift(m, k, _NEG), _NEG))
        k *= 2
    m = jnp.where(first, jnp.maximum(m, ch_ref[...]), m)

    canchor_ref[...] = anchors[blk_n - 1:blk_n, :]
    ch_ref[...] = m[blk_n - 1:blk_n, :]

    est0 = estart_ref[b]

    def ext_body(w, carry):
        est = est0 + w * win
        ends = ends_ref[pl.ds(est, win), :]
        e = (ends == pos_i).astype(jnp.float32)
        vals = jax.lax.dot_general(
            e, m, (((1,), (0,)), ((), ())),
            preferred_element_type=jnp.float32)
        acc_ref[pl.ds(est, win), :] += vals
        return carry

    jax.lax.fori_loop(0, new_ref[b], ext_body, 0)


def _head_kernel(acc_ref, mask_ref, wcr_ref, bcr_ref, out_ref):
    feats = acc_ref[...] * mask_ref[...]
    out_ref[...] = jax.lax.dot_general(
        feats, wcr_ref[...], (((1,), (0,)), ((), ())),
        preferred_element_type=jnp.float32) + bcr_ref[...]


def kernel(points, proposals, cu_seqlens, W1, b1, W2, b2, Wc, bc, Wr, br):
    total = points.shape[0]
    num_props = proposals.shape[0]
    blk_n = 2048 if total % 2048 == 0 else 512
    nb = total // blk_n
    win = min(128, num_props)
    acc_rows = num_props + win

    cu = cu_seqlens.astype(jnp.int32)

    center = proposals[:, :3]
    corner = center + 0.5 * proposals[:, 3:6]
    prop16 = jnp.concatenate(
        [jnp.pad(center, ((0, 0), (0, 1))),
         jnp.pad(corner, ((0, 0), (0, 1))),
         jnp.ones((num_props, 1), jnp.float32),
         jnp.zeros((num_props, 7), jnp.float32)],
        axis=1)

    cu0 = cu[:-1]
    cu1 = cu[1:]
    ends = cu1 - 1
    maskf = (cu1 - cu0 >= _MIN_PTS).astype(jnp.float32)[:, None]

    pad = acc_rows - num_props
    cu0p = jnp.pad(cu0, (0, pad), constant_values=-5)[:, None]
    cu1p = jnp.pad(cu1, (0, pad), constant_values=-5)[:, None]
    endsp = jnp.pad(ends, (0, pad), constant_values=-5)[:, None]
    prop16p = jnp.pad(prop16, ((0, pad), (0, 0)))

    blk = jnp.arange(nb, dtype=jnp.int32) * blk_n
    kstart = jnp.searchsorted(cu0, blk, side='left').astype(jnp.int32)
    kend = jnp.searchsorted(cu0, blk + blk_n, side='left').astype(jnp.int32)
    kstart_al = (kstart // 8) * 8
    nkw = -(-(kend - kstart_al) // win)
    estart = jnp.searchsorted(ends, blk, side='left').astype(jnp.int32)
    eend = jnp.searchsorted(ends, blk + blk_n, side='left').astype(jnp.int32)
    estart_al = (estart // 8) * 8
    new = -(-(eend - estart_al) // win)

    b1r = b1[None, :]
    b2r = b2[None, :]

    smem = pl.BlockSpec(memory_space=pltpu.SMEM)
    full = lambda shape: pl.BlockSpec(shape, lambda i: (0, 0))

    import functools
    acc = pl.pallas_call(
        functools.partial(_fwd_kernel, blk_n=blk_n, win=win),
        grid=(nb,),
        in_specs=[
            smem, smem, smem, smem,
            pl.BlockSpec((blk_n, 3), lambda i: (i, 0)),
            full((acc_rows, 16)),
            full((acc_rows, 1)), full((acc_rows, 1)), full((acc_rows, 1)),
            full((3, 64)), full((1, 64)), full((64, 64)), full((1, 64)),
        ],
        out_specs=full((acc_rows, 128)),
        out_shape=jax.ShapeDtypeStruct((acc_rows, 128), jnp.float32),
        scratch_shapes=[
            pltpu.VMEM((1, 8), jnp.float32),
            pltpu.VMEM((1, 128), jnp.float32),
        ],
    )(kstart_al, nkw, estart_al, new,
      points, prop16p, cu0p, cu1p, endsp,
      W1, b1r, W2, b2r)

    wcr = jnp.pad(jnp.concatenate([Wc, Wr], axis=1), ((0, 0), (0, 1)))
    bcr = jnp.pad(jnp.concatenate([bc, br]), (0, 1))[None, :]

    out = pl.pallas_call(
        _head_kernel,
        grid=(1,),
        in_specs=[full((num_props, 128)), full((num_props, 1)),
                  full((128, 8)), full((1, 8))],
        out_specs=full((num_props, 8)),
        out_shape=jax.ShapeDtypeStruct((num_props, 8), jnp.float32),
    )(acc, maskf, wcr, bcr)

    return out[:, 0:1], out[:, 1:7]


# speedup vs baseline: 1.5586x; 1.5586x over previous
import jax
import jax.numpy as jnp
from jax.experimental import pallas as pl
from jax.experimental.pallas import tpu as pltpu

_MIN_PTS = 4
_NEG = -3.0e38


def _shift(x, k, fill):
    return jnp.concatenate(
        [jnp.full((k,) + x.shape[1:], fill, x.dtype), x[:-k]], axis=0)


def _fwd_kernel(kstart_ref, nkw_ref, estart_ref, new_ref,
                pts_ref, prop_ref, cu0_ref, cu1_ref, ends_ref,
                w1_ref, b1_ref, w2_ref, b2_ref,
                acc_ref, canchor_ref, ch_ref, *, blk_n, win):
    b = pl.program_id(0)

    @pl.when(b == 0)
    def _init():
        acc_ref[...] = jnp.zeros(acc_ref.shape, acc_ref.dtype)
        canchor_ref[...] = jnp.zeros(canchor_ref.shape, canchor_ref.dtype)
        ch_ref[...] = jnp.full(ch_ref.shape, _NEG, ch_ref.dtype)

    s = b * blk_n
    pts = pts_ref[...]
    pos_i = s + jax.lax.broadcasted_iota(jnp.int32, (1, blk_n), 1)

    kst0 = kstart_ref[b]
    ones_w = jnp.ones((win, 1), jnp.float32)

    def place_body(w, carry):
        placed, cnt = carry
        kst = kst0 + w * win
        cu0 = cu0_ref[pl.ds(kst, win), :]
        cu1 = cu1_ref[pl.ds(kst, win), :]
        hitf = jnp.where(cu0 == pos_i, 1.0, 0.0)
        t = jnp.where(cu1 > pos_i, hitf, 0.0)
        pw = prop_ref[pl.ds(kst, win), :]
        placed = placed + jax.lax.dot_general(
            t, pw, (((0,), (0,)), ((), ())),
            preferred_element_type=jnp.float32)
        cnt = cnt + jax.lax.dot_general(
            hitf, ones_w, (((0,), (0,)), ((), ())),
            preferred_element_type=jnp.float32)
        return placed, cnt

    a, c = jax.lax.fori_loop(
        0, nkw_ref[b], place_body,
        (jnp.zeros((blk_n, 8), jnp.float32),
         jnp.zeros((blk_n, 1), jnp.float32)))

    masks = []
    k = 1
    while k < blk_n:
        mask = c == 0.0
        masks.append(mask)
        a = a + jnp.where(mask, _shift(a, k, 0.0), 0.0)
        c = c + _shift(c, k, 0.0)
        k *= 2

    first = c == 0.0
    anchors = a + jnp.where(first, 1.0, 0.0) * canchor_ref[...]

    relc = pts - anchors[:, 0:3]
    relo = pts - anchors[:, 4:7]

    w1 = w1_ref[...]
    w2 = w2_ref[...]
    b1 = b1_ref[...]
    b2 = b2_ref[...]

    def mlp(rel):
        h = jnp.maximum(jax.lax.dot_general(
            rel, w1, (((1,), (0,)), ((), ())),
            preferred_element_type=jnp.float32) + b1, 0.0)
        return jnp.maximum(jax.lax.dot_general(
            h, w2, (((1,), (0,)), ((), ())),
            preferred_element_type=jnp.float32) + b2, 0.0)

    hcat = jnp.concatenate([mlp(relc), mlp(relo)], axis=1)

    m = hcat
    k = 1
    for mask in masks:
        m = jnp.maximum(m, jnp.where(mask, _shift(m, k, _NEG), _NEG))
        k *= 2
    m = jnp.where(first, jnp.maximum(m, ch_ref[...]), m)

    canchor_ref[...] = anchors[blk_n - 1:blk_n, :]
    ch_ref[...] = m[blk_n - 1:blk_n, :]

    est0 = estart_ref[b]

    def ext_body(w, carry):
        est = est0 + w * win
        ends = ends_ref[pl.ds(est, win), :]
        e = (ends == pos_i).astype(jnp.float32)
        vals = jax.lax.dot_general(
            e, m, (((1,), (0,)), ((), ())),
            preferred_element_type=jnp.float32)
        acc_ref[pl.ds(est, win), :] += vals
        return carry

    jax.lax.fori_loop(0, new_ref[b], ext_body, 0)


def _head_kernel(acc_ref, mask_ref, wcr_ref, bcr_ref, out_ref):
    feats = acc_ref[...] * mask_ref[...]
    out_ref[...] = jax.lax.dot_general(
        feats, wcr_ref[...], (((1,), (0,)), ((), ())),
        preferred_element_type=jnp.float32) + bcr_ref[...]


def kernel(points, proposals, cu_seqlens, W1, b1, W2, b2, Wc, bc, Wr, br):
    total = points.shape[0]
    num_props = proposals.shape[0]
    blk_n = 2048 if total % 2048 == 0 else 512
    nb = total // blk_n
    win = min(128, num_props)
    acc_rows = num_props + win

    cu = cu_seqlens.astype(jnp.int32)

    center = proposals[:, :3]
    corner = center + 0.5 * proposals[:, 3:6]
    prop8 = jnp.concatenate(
        [jnp.pad(center, ((0, 0), (0, 1))), jnp.pad(corner, ((0, 0), (0, 1)))],
        axis=1)

    cu0 = cu[:-1]
    cu1 = cu[1:]
    ends = cu1 - 1
    maskf = (cu1 - cu0 >= _MIN_PTS).astype(jnp.float32)[:, None]

    pad = acc_rows - num_props
    cu0p = jnp.pad(cu0, (0, pad), constant_values=-5)[:, None]
    cu1p = jnp.pad(cu1, (0, pad), constant_values=-5)[:, None]
    endsp = jnp.pad(ends, (0, pad), constant_values=-5)[:, None]
    prop8p = jnp.pad(prop8, ((0, pad), (0, 0)))

    blk = jnp.arange(nb, dtype=jnp.int32) * blk_n
    kstart = jnp.searchsorted(cu0, blk, side='left').astype(jnp.int32)
    kend = jnp.searchsorted(cu0, blk + blk_n, side='left').astype(jnp.int32)
    kstart_al = (kstart // 8) * 8
    nkw = -(-(kend - kstart_al) // win)
    estart = jnp.searchsorted(ends, blk, side='left').astype(jnp.int32)
    eend = jnp.searchsorted(ends, blk + blk_n, side='left').astype(jnp.int32)
    estart_al = (estart // 8) * 8
    new = -(-(eend - estart_al) // win)

    b1r = b1[None, :]
    b2r = b2[None, :]

    smem = pl.BlockSpec(memory_space=pltpu.SMEM)
    full = lambda shape: pl.BlockSpec(shape, lambda i: (0, 0))

    import functools
    acc = pl.pallas_call(
        functools.partial(_fwd_kernel, blk_n=blk_n, win=win),
        grid=(nb,),
        in_specs=[
            smem, smem, smem, smem,
            pl.BlockSpec((blk_n, 3), lambda i: (i, 0)),
            full((acc_rows, 8)),
            full((acc_rows, 1)), full((acc_rows, 1)), full((acc_rows, 1)),
            full((3, 64)), full((1, 64)), full((64, 64)), full((1, 64)),
        ],
        out_specs=full((acc_rows, 128)),
        out_shape=jax.ShapeDtypeStruct((acc_rows, 128), jnp.float32),
        scratch_shapes=[
            pltpu.VMEM((1, 8), jnp.float32),
            pltpu.VMEM((1, 128), jnp.float32),
        ],
    )(kstart_al, nkw, estart_al, new,
      points, prop8p, cu0p, cu1p, endsp,
      W1, b1r, W2, b2r)

    wcr = jnp.pad(jnp.concatenate([Wc, Wr], axis=1), ((0, 0), (0, 1)))
    bcr = jnp.pad(jnp.concatenate([bc, br]), (0, 1))[None, :]

    out = pl.pallas_call(
        _head_kernel,
        grid=(1,),
        in_specs=[full((num_props, 128)), full((num_props, 1)),
                  full((128, 8)), full((1, 8))],
        out_specs=full((num_props, 8)),
        out_shape=jax.ShapeDtypeStruct((num_props, 8), jnp.float32),
    )(acc, maskf, wcr, bcr)

    return out[:, 0:1], out[:, 1:7]
